# Initial kernel scaffold; baseline (speedup 1.0000x reference)
import jax
import jax.numpy as jnp
from jax.experimental import pallas as pl


def kernel(x_h, x_g, edge_index, edge_attr, u, batch_e, W1, b1, W2, b2):
    raise NotImplementedError("write your pallas kernel here")


# speedup vs baseline: 11.8555x; 11.8555x over previous
import functools

import jax
import jax.numpy as jnp
from jax import lax
from jax.experimental import pallas as pl
from jax.experimental.pallas import tpu as pltpu
from jax.experimental.pallas import tpu_sc as plsc

_N = 10000
_E = 320000
_N_H = 128
_N_G = 128
_N_X = 16
_N_U = 16
_G = 16

_NC = 2
_NS = 16
_NW = _NC * _NS
_L = 16
_EPW = _E // _NW

_EBLK = 40000


def _node_proj_body(xh_ref, xg_ref, u_ref, w1h_ref, w1g_ref, w1u_ref,
                    w2_ref, b2_ref, a_ref, b_ref, d_ref, w2v_ref, b2v_ref):
    a_ref[...] = jnp.dot(xh_ref[...], w1h_ref[...],
                         preferred_element_type=jnp.float32)
    b_ref[...] = jnp.dot(xg_ref[...], w1g_ref[...],
                         preferred_element_type=jnp.float32)
    d_ref[...] = jnp.dot(u_ref[...], w1u_ref[...],
                         preferred_element_type=jnp.float32)
    ones = jnp.ones((1, _L), jnp.float32)
    w2v_ref[...] = w2_ref[...] * ones
    b2v_ref[...] = b2_ref[...] * ones


def _edge_proj_body(ea_ref, w1x_ref, b1_ref, c_ref):
    w_rep = jnp.concatenate([w1x_ref[...]] * 8, axis=0)
    row = lax.broadcasted_iota(jnp.int32, (128, 8), 0)
    col = lax.broadcasted_iota(jnp.int32, (128, 8), 1)
    bmat = jnp.where(row // 16 == col, w_rep, 0.0)
    c_ref[...] = jnp.dot(ea_ref[...], bmat,
                         preferred_element_type=jnp.float32) + b1_ref[...]


_node_proj = pl.pallas_call(
    _node_proj_body,
    out_shape=(
        jax.ShapeDtypeStruct((_N, 1), jnp.float32),
        jax.ShapeDtypeStruct((_N, 1), jnp.float32),
        jax.ShapeDtypeStruct((_G, 1), jnp.float32),
        jax.ShapeDtypeStruct((1, _L), jnp.float32),
        jax.ShapeDtypeStruct((1, _L), jnp.float32),
    ),
)

_EROWS = _E // 8
_EBLK_R = 5000

_edge_proj = pl.pallas_call(
    _edge_proj_body,
    grid=(_EROWS // _EBLK_R,),
    in_specs=[
        pl.BlockSpec((_EBLK_R, 128), lambda i: (i, 0)),
        pl.BlockSpec((_N_X, 1), lambda i: (0, 0)),
        pl.BlockSpec((1, 1), lambda i: (0, 0)),
    ],
    out_specs=pl.BlockSpec((_EBLK_R, 8), lambda i: (i, 0)),
    out_shape=jax.ShapeDtypeStruct((_EROWS, 8), jnp.float32),
)


@functools.partial(
    pl.kernel,
    out_type=jax.ShapeDtypeStruct((_E,), jnp.float32),
    mesh=plsc.VectorSubcoreMesh(core_axis_name="c", subcore_axis_name="s"),
    compiler_params=pltpu.CompilerParams(needs_layout_passes=False),
    scratch_types=[
        pltpu.VMEM((_N,), jnp.float32),
        pltpu.VMEM((_N,), jnp.float32),
        pltpu.VMEM((_G,), jnp.float32),
        pltpu.VMEM((_L,), jnp.float32),
        pltpu.VMEM((_L,), jnp.float32),
        pltpu.VMEM((_EPW,), jnp.int32),
        pltpu.VMEM((_EPW,), jnp.int32),
        pltpu.VMEM((_EPW,), jnp.int32),
        pltpu.VMEM((_EPW,), jnp.float32),
        pltpu.VMEM((_EPW,), jnp.float32),
    ],
)
def _sc_combine(a_hbm, b_hbm, d_hbm, w2_hbm, b2_hbm,
                src_hbm, tgt_hbm, bat_hbm, c_hbm, out_hbm,
                a_v, b_v, d_v, w2_v, b2_v,
                src_v, tgt_v, bat_v, c_v, out_v):
    wid = lax.axis_index("s") * _NC + lax.axis_index("c")
    base = wid * _EPW
    pltpu.sync_copy(a_hbm, a_v)
    pltpu.sync_copy(b_hbm, b_v)
    pltpu.sync_copy(d_hbm, d_v)
    pltpu.sync_copy(w2_hbm, w2_v)
    pltpu.sync_copy(b2_hbm, b2_v)
    pltpu.sync_copy(src_hbm.at[pl.ds(base, _EPW)], src_v)
    pltpu.sync_copy(tgt_hbm.at[pl.ds(base, _EPW)], tgt_v)
    pltpu.sync_copy(bat_hbm.at[pl.ds(base, _EPW)], bat_v)
    pltpu.sync_copy(c_hbm.at[pl.ds(base, _EPW)], c_v)

    w2 = w2_v[...]
    b2 = b2_v[...]

    def body(i, carry):
        o = i * _L
        sv = plsc.load_gather(a_v, [src_v[pl.ds(o, _L)]])
        tv = plsc.load_gather(b_v, [tgt_v[pl.ds(o, _L)]])
        gv = plsc.load_gather(d_v, [bat_v[pl.ds(o, _L)]])
        h = sv + tv + gv + c_v[pl.ds(o, _L)]
        h = jnp.where(h >= 0.0, h, h * jnp.float32(0.1))
        out_v[pl.ds(o, _L)] = h * w2 + b2
        return carry

    lax.fori_loop(0, _EPW // _L, body, 0)
    pltpu.sync_copy(out_v, out_hbm.at[pl.ds(base, _EPW)])


def kernel(x_h, x_g, edge_index, edge_attr, u, batch_e, W1, b1, W2, b2):
    src = edge_index[0].astype(jnp.int32)
    tgt = edge_index[1].astype(jnp.int32)
    bat = batch_e.astype(jnp.int32)
    w1h = W1[0:_N_H]
    w1g = W1[_N_H:_N_H + _N_G]
    w1x = W1[_N_H + _N_G:_N_H + _N_G + _N_X]
    w1u = W1[_N_H + _N_G + _N_X:]

    a, b, d, w2v, b2v = _node_proj(x_h, x_g, u, w1h, w1g, w1u,
                                   W2.reshape(1, 1), b2.reshape(1, 1))
    c = _edge_proj(edge_attr.reshape(_EROWS, 128), w1x, b1.reshape(1, 1))

    out = _sc_combine(a.reshape(_N), b.reshape(_N), d.reshape(_G),
                      w2v.reshape(_L), b2v.reshape(_L),
                      src, tgt, bat, c.reshape(_E))
    return out.reshape(_E, 1)


# speedup vs baseline: 25.3475x; 2.1380x over previous
import functools

import jax
import jax.numpy as jnp
from jax import lax
from jax.experimental import pallas as pl
from jax.experimental.pallas import tpu as pltpu
from jax.experimental.pallas import tpu_sc as plsc

_N = 10000
_E = 320000
_N_H = 128
_N_G = 128
_N_X = 16
_G = 16

_NC = 2
_NS = 16
_NW = _NC * _NS
_L = 16
_S = 2560
_NP = 4
_EPW = _S * _NP


def _node_proj_body(xh_ref, xg_ref, u_ref, w1h_ref, w1g_ref, w1x_ref,
                    w1u_ref, b1_ref, w2_ref, b2_ref,
                    a_ref, b_ref, d_ref, wx_ref, w2v_ref, b2v_ref):
    a_ref[...] = jnp.dot(xh_ref[...], w1h_ref[...],
                         preferred_element_type=jnp.float32)
    b_ref[...] = jnp.dot(xg_ref[...], w1g_ref[...],
                         preferred_element_type=jnp.float32)
    d_ref[...] = jnp.dot(u_ref[...], w1u_ref[...],
                         preferred_element_type=jnp.float32) + b1_ref[...]
    ones = jnp.ones((1, _L), jnp.float32)
    wx_ref[...] = w1x_ref[...] * ones
    w2v_ref[...] = w2_ref[...] * ones
    b2v_ref[...] = b2_ref[...] * ones


_node_proj = pl.pallas_call(
    _node_proj_body,
    out_shape=(
        jax.ShapeDtypeStruct((_N, 1), jnp.float32),
        jax.ShapeDtypeStruct((_N, 1), jnp.float32),
        jax.ShapeDtypeStruct((_G, 1), jnp.float32),
        jax.ShapeDtypeStruct((_N_X, _L), jnp.float32),
        jax.ShapeDtypeStruct((1, _L), jnp.float32),
        jax.ShapeDtypeStruct((1, _L), jnp.float32),
    ),
)


@functools.partial(
    pl.kernel,
    out_type=jax.ShapeDtypeStruct((_E,), jnp.float32),
    mesh=plsc.VectorSubcoreMesh(core_axis_name="c", subcore_axis_name="s"),
    compiler_params=pltpu.CompilerParams(needs_layout_passes=False),
    scratch_types=[
        pltpu.VMEM((_N,), jnp.float32),
        pltpu.VMEM((_N,), jnp.float32),
        pltpu.VMEM((_G,), jnp.float32),
        pltpu.VMEM((_N_X, _L), jnp.float32),
        pltpu.VMEM((_L,), jnp.float32),
        pltpu.VMEM((_L,), jnp.float32),
        pltpu.VMEM((_N_X, _S), jnp.float32),
        pltpu.VMEM((_S,), jnp.int32),
        pltpu.VMEM((_S,), jnp.int32),
        pltpu.VMEM((_S,), jnp.int32),
        pltpu.VMEM((_S,), jnp.float32),
    ],
)
def _sc_combine(a_hbm, b_hbm, d_hbm, wx_hbm, w2_hbm, b2_hbm,
                src_hbm, tgt_hbm, bat_hbm, eat_hbm, out_hbm,
                a_v, b_v, d_v, wx_v, w2_v, b2_v,
                col_v, src_v, tgt_v, bat_v, out_v):
    wid = lax.axis_index("s") * _NC + lax.axis_index("c")
    base = wid * _EPW
    pltpu.sync_copy(a_hbm, a_v)
    pltpu.sync_copy(b_hbm, b_v)
    pltpu.sync_copy(d_hbm, d_v)
    pltpu.sync_copy(wx_hbm, wx_v)
    pltpu.sync_copy(w2_hbm, w2_v)
    pltpu.sync_copy(b2_hbm, b2_v)

    w2 = w2_v[...]
    b2 = b2_v[...]
    wx = [wx_v[k, :] for k in range(_N_X)]

    for p in range(_NP):
        pb = base + p * _S

        @pl.when(pb < _E)
        def _one_pass(pb=pb):
            pltpu.sync_copy(src_hbm.at[pl.ds(pb, _S)], src_v)
            pltpu.sync_copy(tgt_hbm.at[pl.ds(pb, _S)], tgt_v)
            pltpu.sync_copy(bat_hbm.at[pl.ds(pb, _S)], bat_v)
            pltpu.sync_copy(eat_hbm.at[:, pl.ds(pb, _S)], col_v)

            def body(i, carry):
                o = i * _L
                acc = (plsc.load_gather(a_v, [src_v[pl.ds(o, _L)]])
                       + plsc.load_gather(b_v, [tgt_v[pl.ds(o, _L)]])
                       + plsc.load_gather(d_v, [bat_v[pl.ds(o, _L)]]))
                terms = [wx[k] * col_v[k, pl.ds(o, _L)]
                         for k in range(_N_X)]
                while len(terms) > 1:
                    terms = [terms[j] + terms[j + 1]
                             for j in range(0, len(terms) - 1, 2)] + (
                                 [terms[-1]] if len(terms) % 2 else [])
                h = acc + terms[0]
                h = jnp.where(h >= 0.0, h, h * jnp.float32(0.1))
                out_v[pl.ds(o, _L)] = h * w2 + b2
                return carry

            lax.fori_loop(0, _S // _L, body, 0)
            pltpu.sync_copy(out_v, out_hbm.at[pl.ds(pb, _S)])


def kernel(x_h, x_g, edge_index, edge_attr, u, batch_e, W1, b1, W2, b2):
    ei = edge_index.astype(jnp.int32)
    src = ei[0]
    tgt = ei[1]
    bat = batch_e.astype(jnp.int32)
    w1h = W1[0:_N_H]
    w1g = W1[_N_H:_N_H + _N_G]
    w1x = W1[_N_H + _N_G:_N_H + _N_G + _N_X]
    w1u = W1[_N_H + _N_G + _N_X:]

    a, b, d, wx, w2v, b2v = _node_proj(
        x_h, x_g, u, w1h, w1g, w1x, w1u,
        b1.reshape(1, 1), W2.reshape(1, 1), b2.reshape(1, 1))

    out = _sc_combine(a.reshape(_N), b.reshape(_N), d.reshape(_G),
                      wx, w2v.reshape(_L), b2v.reshape(_L),
                      src, tgt, bat, edge_attr.T)
    return out.reshape(_E, 1)


# speedup vs baseline: 39.5538x; 1.5605x over previous
import functools

import jax
import jax.numpy as jnp
from jax import lax
from jax.experimental import pallas as pl
from jax.experimental.pallas import tpu as pltpu
from jax.experimental.pallas import tpu_sc as plsc

_N = 10000
_E = 320000
_N_H = 128
_N_G = 128
_N_X = 16
_G = 16

_NC = 2
_NS = 16
_NW = _NC * _NS
_L = 16
_S = 2560
_NP = 4
_EPW = _S * _NP


def _node_proj_body(xh_ref, xg_ref, u_ref, w1_ref, b1_ref, w2_ref, b2_ref,
                    a_ref, b_ref, misc_ref):
    cdims = (((1,), (1,)), ((), ()))
    w1h = w1_ref[pl.ds(0, _N_H)].reshape(1, _N_H)
    w1g = w1_ref[pl.ds(_N_H, _N_G)].reshape(1, _N_G)
    w1u = w1_ref[pl.ds(_N_H + _N_G + _N_X, 16)].reshape(1, 16)
    a_ref[...] = lax.dot_general(
        w1h, xh_ref[...], cdims,
        preferred_element_type=jnp.float32).reshape(_N)
    b_ref[...] = lax.dot_general(
        w1g, xg_ref[...], cdims,
        preferred_element_type=jnp.float32).reshape(_N)
    d = lax.dot_general(w1u, u_ref[...], cdims,
                        preferred_element_type=jnp.float32) + b1_ref[...]
    ones = jnp.ones((1, _L), jnp.float32)
    misc = jnp.concatenate([d, w2_ref[...] * ones, b2_ref[...] * ones],
                           axis=1)
    misc_ref[...] = misc.reshape(3 * _L)


_node_proj = pl.pallas_call(
    _node_proj_body,
    out_shape=(
        jax.ShapeDtypeStruct((_N,), jnp.float32),
        jax.ShapeDtypeStruct((_N,), jnp.float32),
        jax.ShapeDtypeStruct((3 * _L,), jnp.float32),
    ),
)


def _edge_proj_body(eat_ref, w1_ref, c_ref):
    w1x = w1_ref[pl.ds(_N_H + _N_G, _N_X)].reshape(1, _N_X)
    cdims = (((1,), (0,)), ((), ()))
    c_ref[...] = lax.dot_general(
        w1x, eat_ref[...], cdims,
        preferred_element_type=jnp.float32).reshape(_E)


_edge_proj = pl.pallas_call(
    _edge_proj_body,
    out_shape=jax.ShapeDtypeStruct((_E,), jnp.float32),
)


@functools.partial(
    pl.kernel,
    out_type=jax.ShapeDtypeStruct((_E,), jnp.float32),
    mesh=plsc.VectorSubcoreMesh(core_axis_name="c", subcore_axis_name="s"),
    compiler_params=pltpu.CompilerParams(needs_layout_passes=False),
    scratch_types=[
        pltpu.VMEM((_N,), jnp.float32),
        pltpu.VMEM((_N,), jnp.float32),
        pltpu.VMEM((3 * _L,), jnp.float32),
        pltpu.VMEM((2, _S), jnp.int32),
        pltpu.VMEM((_S,), jnp.int32),
        pltpu.VMEM((_S,), jnp.float32),
        pltpu.VMEM((_S,), jnp.float32),
    ],
)
def _sc_combine(a_hbm, b_hbm, misc_hbm, ei_hbm, bat_hbm, c_hbm, out_hbm,
                a_v, b_v, misc_v, st_v, bat_v, c_v, out_v):
    wid = lax.axis_index("s") * _NC + lax.axis_index("c")
    base = wid * _EPW
    pltpu.sync_copy(a_hbm, a_v)
    pltpu.sync_copy(b_hbm, b_v)
    pltpu.sync_copy(misc_hbm, misc_v)

    w2 = misc_v[pl.ds(_L, _L)]
    b2 = misc_v[pl.ds(2 * _L, _L)]

    for p in range(_NP):
        pb = base + p * _S

        @pl.when(pb < _E)
        def _one_pass(pb=pb):
            pltpu.sync_copy(ei_hbm.at[:, pl.ds(pb, _S)], st_v)
            pltpu.sync_copy(bat_hbm.at[pl.ds(pb, _S)], bat_v)
            pltpu.sync_copy(c_hbm.at[pl.ds(pb, _S)], c_v)

            def body(i, carry):
                o = i * _L
                h = (plsc.load_gather(a_v, [st_v[0, pl.ds(o, _L)]])
                     + plsc.load_gather(b_v, [st_v[1, pl.ds(o, _L)]])
                     + plsc.load_gather(misc_v, [bat_v[pl.ds(o, _L)]])
                     + c_v[pl.ds(o, _L)])
                h = jnp.where(h >= 0.0, h, h * jnp.float32(0.1))
                out_v[pl.ds(o, _L)] = h * w2 + b2
                return carry

            lax.fori_loop(0, _S // _L, body, 0)
            pltpu.sync_copy(out_v, out_hbm.at[pl.ds(pb, _S)])


def kernel(x_h, x_g, edge_index, edge_attr, u, batch_e, W1, b1, W2, b2):
    ei = edge_index.astype(jnp.int32)
    bat = batch_e.astype(jnp.int32)
    w1 = W1.reshape(_N_H + _N_G + _N_X + 16)

    a, b, misc = _node_proj(x_h, x_g, u, w1, b1.reshape(1, 1),
                            W2.reshape(1, 1), b2.reshape(1, 1))
    c = _edge_proj(edge_attr.T, w1)
    out = _sc_combine(a, b, misc, ei, bat, c)
    return out.reshape(_E, 1)
